# baseline (speedup 1.0000x reference)
import functools

import jax
import jax.numpy as jnp
from jax import lax
from jax.experimental import pallas as pl
from jax.experimental.pallas import tpu as pltpu
from jax.experimental.pallas import tpu_sc as plsc

VOCAB = 100000
EMBED = 32
B = 1024
L = 200

NC = 2
NS = 16
NW = NC * NS
NH = 2
BH = B // NH
BPW = BH // NW
CHUNK = 100
CPR = L // CHUNK
NCHUNK = BPW * CPR


def _make_pool():
    mesh = plsc.VectorSubcoreMesh(core_axis_name="c", subcore_axis_name="s")

    @functools.partial(
        pl.kernel,
        mesh=mesh,
        compiler_params=pltpu.CompilerParams(use_tc_tiling_on_sc=False),
        out_type=jax.ShapeDtypeStruct((BH, EMBED), jnp.float32),
        scratch_types=[
            pltpu.VMEM((NCHUNK, CHUNK), jnp.int32),
            pltpu.VMEM((CHUNK, EMBED), jnp.float32),
            pltpu.VMEM((CHUNK, EMBED), jnp.float32),
            pltpu.VMEM((BPW, EMBED), jnp.float32),
            pltpu.SemaphoreType.DMA,
            pltpu.SemaphoreType.DMA,
        ],
    )
    def pool(ids_hbm, table_hbm, out_hbm, idx_v, rows_a, rows_b, out_v,
             sem_a, sem_b):
        wid = lax.axis_index("s") * NC + lax.axis_index("c")
        pltpu.sync_copy(ids_hbm.at[wid], idx_v)
        inv_l = jnp.float32(1.0 / L)

        def acc_chunk(rows_v, accs):
            def tok_body(t, accs2):
                b0, b1 = accs2
                return (b0 + rows_v[t, pl.ds(0, 16)],
                        b1 + rows_v[t, pl.ds(16, 16)])

            return lax.fori_loop(0, CHUNK, tok_body, accs, unroll=10)

        pltpu.async_copy(table_hbm.at[idx_v.at[0]], rows_a, sem_a)

        def row_body(i, carry):
            pltpu.async_copy(table_hbm.at[idx_v.at[2 * i + 1]], rows_b, sem_b)
            pltpu.make_async_copy(table_hbm.at[idx_v.at[0]], rows_a,
                                  sem_a).wait()
            z = jnp.zeros((16,), jnp.float32)
            a0, a1 = acc_chunk(rows_a, (z, z))

            @pl.when(i < BPW - 1)
            def _():
                pltpu.async_copy(table_hbm.at[idx_v.at[2 * i + 2]], rows_a,
                                 sem_a)

            pltpu.make_async_copy(table_hbm.at[idx_v.at[0]], rows_b,
                                  sem_b).wait()
            a0, a1 = acc_chunk(rows_b, (a0, a1))
            out_v[i, pl.ds(0, 16)] = a0 * inv_l
            out_v[i, pl.ds(16, 16)] = a1 * inv_l
            return carry

        lax.fori_loop(0, BPW, row_body, 0)
        pltpu.sync_copy(out_v, out_hbm.at[pl.ds(wid * BPW, BPW)])

    return pool


_pool = _make_pool()

BN = 4096
GRID_N = (VOCAB + BN - 1) // BN


def _mm_body(w_ref, b_ref, x_ref, o_ref):
    lhs = jnp.concatenate([w_ref[...], b_ref[...]], axis=0)
    rhs = jnp.concatenate(
        [x_ref[...], jnp.ones((BH, 1), jnp.float32)], axis=1
    )
    o_ref[...] = lax.dot_general(
        lhs, rhs, (((0,), (1,)), ((), ())),
        preferred_element_type=jnp.float32,
    )


def _mm_first_body(w_ref, b_ref, x_ref, o_ref):
    _mm_body(w_ref, b_ref, x_ref, o_ref)


def _matmul_half(h, w, b2, pooled_h, prev=None):
    in_specs = [
        pl.BlockSpec((EMBED, BN), lambda n: (0, n)),
        pl.BlockSpec((1, BN), lambda n: (0, n)),
        pl.BlockSpec((BH, EMBED), lambda n: (0, 0)),
    ]
    args = [w, b2, pooled_h]
    kwargs = {}
    if prev is not None:
        in_specs.append(pl.BlockSpec(memory_space=pltpu.MemorySpace.HBM))
        args.append(prev)
        kwargs["input_output_aliases"] = {3: 0}
        body = lambda w_ref, b_ref, x_ref, p_ref, o_ref: _mm_body(
            w_ref, b_ref, x_ref, o_ref)
    else:
        body = _mm_first_body
    return pl.pallas_call(
        body,
        grid=(GRID_N,),
        in_specs=in_specs,
        out_specs=pl.BlockSpec((BN, BH), lambda n: (n, h)),
        out_shape=jax.ShapeDtypeStruct((VOCAB, B), jnp.float32),
        **kwargs,
    )(*args)


def kernel(input_ids, embed_table, W, b):
    b2 = b.reshape(1, VOCAB)
    ids4 = input_ids.reshape(NH, NW, NCHUNK, CHUNK)
    pooled0 = _pool(ids4[0], embed_table)
    pooled1 = _pool(ids4[1], embed_table)
    out = _matmul_half(0, W, b2, pooled0)
    out = _matmul_half(1, W, b2, pooled1, prev=out)
    return jnp.transpose(out)[:, None, :]


# speedup vs baseline: 1.0533x; 1.0533x over previous
import functools

import jax
import jax.numpy as jnp
from jax import lax
from jax.experimental import pallas as pl
from jax.experimental.pallas import tpu as pltpu
from jax.experimental.pallas import tpu_sc as plsc

VOCAB = 100000
EMBED = 32
B = 1024
L = 200

NC = 2
NS = 16
NW = NC * NS
BPW = B // NW
CHUNK = 100
CPR = L // CHUNK
NCHUNK = BPW * CPR


def _make_pool():
    mesh = plsc.VectorSubcoreMesh(core_axis_name="c", subcore_axis_name="s")

    @functools.partial(
        pl.kernel,
        mesh=mesh,
        compiler_params=pltpu.CompilerParams(use_tc_tiling_on_sc=False),
        out_type=jax.ShapeDtypeStruct((B, EMBED), jnp.float32),
        scratch_types=[
            pltpu.VMEM((NCHUNK, CHUNK), jnp.int32),
            pltpu.VMEM((CHUNK, EMBED), jnp.float32),
            pltpu.VMEM((CHUNK, EMBED), jnp.float32),
            pltpu.VMEM((BPW, EMBED), jnp.float32),
            pltpu.SemaphoreType.DMA,
            pltpu.SemaphoreType.DMA,
        ],
    )
    def pool(ids_hbm, table_hbm, out_hbm, idx_v, rows_a, rows_b, out_v,
             sem_a, sem_b):
        wid = lax.axis_index("s") * NC + lax.axis_index("c")
        pltpu.sync_copy(ids_hbm.at[wid], idx_v)
        inv_l = jnp.float32(1.0 / L)

        def acc_chunk(rows_v, accs):
            def tok_body(t, accs2):
                b0, b1 = accs2
                return (b0 + rows_v[t, pl.ds(0, 16)],
                        b1 + rows_v[t, pl.ds(16, 16)])

            return lax.fori_loop(0, CHUNK, tok_body, accs, unroll=10)

        pltpu.async_copy(table_hbm.at[idx_v.at[0]], rows_a, sem_a)

        def row_body(i, carry):
            pltpu.async_copy(table_hbm.at[idx_v.at[2 * i + 1]], rows_b, sem_b)
            pltpu.make_async_copy(table_hbm.at[idx_v.at[0]], rows_a,
                                  sem_a).wait()
            z = jnp.zeros((16,), jnp.float32)
            a0, a1 = acc_chunk(rows_a, (z, z))

            @pl.when(i < BPW - 1)
            def _():
                pltpu.async_copy(table_hbm.at[idx_v.at[2 * i + 2]], rows_a,
                                 sem_a)

            pltpu.make_async_copy(table_hbm.at[idx_v.at[0]], rows_b,
                                  sem_b).wait()
            a0, a1 = acc_chunk(rows_b, (a0, a1))
            out_v[i, pl.ds(0, 16)] = a0 * inv_l
            out_v[i, pl.ds(16, 16)] = a1 * inv_l
            return carry

        lax.fori_loop(0, BPW, row_body, 0)
        pltpu.sync_copy(out_v, out_hbm.at[pl.ds(wid * BPW, BPW)])

    return pool


_pool = _make_pool()

BN = 4096
GRID_N = (VOCAB + BN - 1) // BN


def _mm_body(w_ref, b_ref, x_ref, o_ref):
    lhs = jnp.concatenate([w_ref[...], b_ref[...]], axis=0)
    rhs = jnp.concatenate(
        [x_ref[...], jnp.ones((B, 1), jnp.float32)], axis=1
    )
    o_ref[...] = lax.dot_general(
        lhs, rhs, (((0,), (1,)), ((), ())),
        preferred_element_type=jnp.float32,
    )


def _matmul_t(w, b2, pooled):
    return pl.pallas_call(
        _mm_body,
        grid=(GRID_N,),
        in_specs=[
            pl.BlockSpec((EMBED, BN), lambda n: (0, n)),
            pl.BlockSpec((1, BN), lambda n: (0, n)),
            pl.BlockSpec((B, EMBED), lambda n: (0, 0)),
        ],
        out_specs=pl.BlockSpec((BN, B), lambda n: (n, 0)),
        out_shape=jax.ShapeDtypeStruct((VOCAB, B), jnp.float32),
    )(w, b2, pooled)


def kernel(input_ids, embed_table, W, b):
    ids3 = input_ids.reshape(NW, NCHUNK, CHUNK)
    pooled = _pool(ids3, embed_table)
    logits_t = _matmul_t(W, b.reshape(1, VOCAB), pooled)
    return jnp.transpose(logits_t)[:, None, :]
